# baseline (speedup 1.0000x reference)
import functools
import math

import jax
import jax.numpy as jnp
from jax import lax
from jax.experimental import pallas as pl
from jax.experimental.pallas import tpu as pltpu
from jax.experimental.pallas import tpu_sc as plsc

N = 10000
E = 160000
H = 32
NCORES = 2
NSUB = 16
NW = NCORES * NSUB
CHUNK = 128
NCHUNK = E // CHUNK
RB = 624
TAIL = N - RB * NSUB


@functools.cache
def _sc_mesh():
    return plsc.VectorSubcoreMesh(
        core_axis_name="c", subcore_axis_name="s", num_cores=NCORES, num_subcores=NSUB
    )


def _slab_copy(src, dst, sid):
    r0 = sid * RB
    pltpu.sync_copy(src.at[pl.ds(r0, RB)], dst.at[pl.ds(r0, RB)])

    @pl.when(sid == NSUB - 1)
    def _():
        pltpu.sync_copy(src.at[pl.ds(RB * NSUB, TAIL)], dst.at[pl.ds(RB * NSUB, TAIL)])


def _prep_body(x_ref, w_ref, ss_ref, sd_ref, h_ref, ap_ref, l_ref, acc_ref):
    hb = jnp.dot(x_ref[...], w_ref[...], preferred_element_type=jnp.float32)
    h_ref[...] = hb
    asb = jnp.dot(hb, ss_ref[...], preferred_element_type=jnp.float32)
    adb = jnp.dot(hb, sd_ref[...], preferred_element_type=jnp.float32)
    bn = asb.shape[0]
    ap_ref[...] = jnp.concatenate(
        [asb, adb, jnp.zeros((bn, 128 - 2 * H), jnp.float32)], axis=-1
    )
    i = pl.program_id(0)
    nb = pl.num_programs(0)
    ms = jnp.max(asb, axis=0)[None, :]
    md = jnp.max(adb, axis=0)[None, :]
    cur = jnp.broadcast_to(jnp.concatenate([ms, md], axis=-1), (8, 2 * H))

    @pl.when(i == 0)
    def _():
        acc_ref[...] = cur

    @pl.when(i > 0)
    def _():
        acc_ref[...] = jnp.maximum(acc_ref[...], cur)

    @pl.when(i == nb - 1)
    def _():
        s = acc_ref[:, :H] + acc_ref[:, H:]
        l_ref[...] = jnp.where(s >= 0.0, s, 0.2 * s)


def _prep(xin, w, ss, sd):
    n, k = xin.shape
    hc = w.shape[1]
    bn = 1000
    return pl.pallas_call(
        _prep_body,
        grid=(n // bn,),
        in_specs=[
            pl.BlockSpec((bn, k), lambda i: (i, 0)),
            pl.BlockSpec((k, hc), lambda i: (0, 0)),
            pl.BlockSpec((hc, H), lambda i: (0, 0)),
            pl.BlockSpec((hc, H), lambda i: (0, 0)),
        ],
        out_specs=[
            pl.BlockSpec((bn, hc), lambda i: (i, 0)),
            pl.BlockSpec((bn, 128), lambda i: (i, 0)),
            pl.BlockSpec((8, H), lambda i: (0, 0)),
        ],
        out_shape=[
            jax.ShapeDtypeStruct((n, hc), jnp.float32),
            jax.ShapeDtypeStruct((n, 128), jnp.float32),
            jax.ShapeDtypeStruct((8, H), jnp.float32),
        ],
        scratch_shapes=[pltpu.VMEM((8, 2 * H), jnp.float32)],
    )(xin, w, ss, sd)


def _att_body(
    j_hbm, i_hbm, ap_hbm, l_hbm, z_hbm,
    ex_hbm, den0_hbm, den1_hbm,
    jbuf, ibuf, apj, api, exb, lb, den_sh, sem,
):
    cid0 = lax.axis_index("c")
    sid = lax.axis_index("s")
    wid = cid0 * NSUB + sid
    _slab_copy(z_hbm, den_sh, sid)
    pltpu.sync_copy(l_hbm, lb)
    plsc.subcore_barrier()
    l0 = lb[0, 0:16]
    l1 = lb[0, 16:32]
    zero = jnp.zeros((16,), jnp.float32)

    def zrow(k, c2):
        for s in range(2, 8):
            exb[k, s * 16:(s + 1) * 16] = zero
        return c2

    lax.fori_loop(0, CHUNK, zrow, 0)

    def chunk_body(t, carry):
        base = (wid + t * NW) * CHUNK
        pltpu.sync_copy(j_hbm.at[pl.ds(base, CHUNK)], jbuf)
        pltpu.sync_copy(i_hbm.at[pl.ds(base, CHUNK)], ibuf)
        pltpu.async_copy(ap_hbm.at[jbuf], apj, sem).wait()
        pltpu.async_copy(ap_hbm.at[ibuf], api, sem).wait()

        def row(k, c2):
            s0 = apj[k, 0:16] + api[k, 32:48]
            s1 = apj[k, 16:32] + api[k, 48:64]
            exb[k, 0:16] = jnp.exp(jnp.where(s0 >= 0.0, s0, 0.2 * s0) - l0)
            exb[k, 16:32] = jnp.exp(jnp.where(s1 >= 0.0, s1, 0.2 * s1) - l1)
            return c2

        lax.fori_loop(0, CHUNK, row, 0)
        pltpu.sync_copy(exb, ex_hbm.at[pl.ds(base, CHUNK)])
        pltpu.sync_copy(exb, den_sh.at[ibuf], add=True)
        return carry

    nt = (NCHUNK - wid + NW - 1) // NW
    lax.fori_loop(0, nt, chunk_body, 0)
    plsc.subcore_barrier()

    @pl.when(cid0 == 0)
    def _():
        _slab_copy(den_sh, den0_hbm, sid)

    @pl.when(cid0 == 1)
    def _():
        _slab_copy(den_sh, den1_hbm, sid)


def _att(j, i, attpack, lmax, zer128):
    f = pl.kernel(
        _att_body,
        out_type=[
            jax.ShapeDtypeStruct((E, 128), jnp.float32),
            jax.ShapeDtypeStruct((N, 128), jnp.float32),
            jax.ShapeDtypeStruct((N, 128), jnp.float32),
        ],
        mesh=_sc_mesh(),
        compiler_params=pltpu.CompilerParams(needs_layout_passes=False),
        scratch_types=[
            pltpu.VMEM((CHUNK,), jnp.int32),
            pltpu.VMEM((CHUNK,), jnp.int32),
            pltpu.VMEM((CHUNK, 128), jnp.float32),
            pltpu.VMEM((CHUNK, 128), jnp.float32),
            pltpu.VMEM((CHUNK, 128), jnp.float32),
            pltpu.VMEM((8, H), jnp.float32),
            pltpu.VMEM_SHARED((N, 128), jnp.float32),
            pltpu.SemaphoreType.DMA,
        ],
    )
    return f(j, i, attpack, lmax, zer128)


def _denpack_body(d0_ref, d1_ref, dp_ref):
    dp_ref[...] = d0_ref[...] + d1_ref[...]


def _denpack(d0, d1):
    bn = 1000
    return pl.pallas_call(
        _denpack_body,
        grid=(N // bn,),
        in_specs=[
            pl.BlockSpec((bn, 128), lambda i: (i, 0)),
            pl.BlockSpec((bn, 128), lambda i: (i, 0)),
        ],
        out_specs=pl.BlockSpec((bn, 128), lambda i: (i, 0)),
        out_shape=jax.ShapeDtypeStruct((N, 128), jnp.float32),
    )(d0, d1)


GROWS = 64
ACH = 64
NACH = E // ACH
CA = 64


def _ig16(val):
    return jnp.zeros((16,), jnp.int32) + val


def _agg_body(
    rstride,
    j_hbm, i_hbm, ex_hbm, denp_hbm, h_hbm, z_hbm,
    out0_hbm, out1_hbm,
    jbuf, ibuf, ewb, dmb, idx0, idx1, gb0, gb1, acc_sh, semd, sem0, sem1,
):
    cid0 = lax.axis_index("c")
    sid = lax.axis_index("s")
    wid = cid0 * NSUB + sid
    _slab_copy(z_hbm, acc_sh, sid)
    plsc.subcore_barrier()
    iota = lax.iota(jnp.int32, 16)
    halves = rstride // 16
    rowspe = 16 * halves
    eps = GROWS // rowspe
    nsubs = ACH // eps
    zero = jnp.zeros((16,), jnp.float32)
    idxs = [idx0, idx1]
    gbs = [gb0, gb1]
    sems = [sem0, sem1]

    def build(q, ib):
        for e in range(eps):
            js = plsc.load_gather(jbuf, [_ig16(q * eps + e)])
            b0 = js * rstride + iota
            ib[pl.ds(e * rowspe, 16)] = b0
            if halves == 2:
                ib[pl.ds(e * rowspe + 16, 16)] = b0 + 16

    def fma(q, gb):
        for e in range(eps):
            k = q * eps + e
            kv = _ig16(k)
            acc = [zero] * 8
            for p in range(16):
                ws0 = plsc.load_gather(ewb, [kv, _ig16(2 * p)])
                ws1 = plsc.load_gather(ewb, [kv, _ig16(2 * p + 1)])
                ra = e * rowspe + p
                for s in range(4):
                    acc[s] = acc[s] + ws0 * gb[ra, s * 16:(s + 1) * 16]
                    acc[s] = acc[s] + ws1 * gb[ra, CA + s * 16:CA + (s + 1) * 16]
                if halves == 2:
                    rb = ra + 16
                    for s in range(4):
                        acc[4 + s] = acc[4 + s] + ws0 * gb[rb, s * 16:(s + 1) * 16]
                        acc[4 + s] = acc[4 + s] + ws1 * gb[rb, CA + s * 16:CA + (s + 1) * 16]
            for s in range(8):
                dmb[k, s * 16:(s + 1) * 16] = acc[s]

    def chunk_body(t, carry):
        base = (wid + t * NW) * ACH
        pltpu.sync_copy(j_hbm.at[pl.ds(base, ACH)], jbuf)
        pltpu.sync_copy(i_hbm.at[pl.ds(base, ACH)], ibuf)
        den_cp = pltpu.async_copy(denp_hbm.at[ibuf], dmb, semd)
        pltpu.sync_copy(ex_hbm.at[pl.ds(base, ACH)], ewb)
        den_cp.wait()

        def wrow(k, c2):
            ewb[k, 0:16] = ewb[k, 0:16] / (dmb[k, 0:16] + 1e-16)
            ewb[k, 16:32] = ewb[k, 16:32] / (dmb[k, 16:32] + 1e-16)
            return c2

        lax.fori_loop(0, ACH, wrow, 0)

        def pair(q2, c3):
            q = q2 * 2
            build(q, idx0)
            cp0 = pltpu.async_copy(h_hbm.at[idx0], gb0, sem0)
            build(q + 1, idx1)
            cp1 = pltpu.async_copy(h_hbm.at[idx1], gb1, sem1)
            cp0.wait()
            fma(q, gb0)
            cp1.wait()
            fma(q + 1, gb1)
            return c3

        lax.fori_loop(0, nsubs // 2, pair, 0)
        pltpu.sync_copy(dmb, acc_sh.at[ibuf], add=True)
        return carry

    nt = (NACH - wid + NW - 1) // NW
    lax.fori_loop(0, nt, chunk_body, 0)
    plsc.subcore_barrier()

    @pl.when(cid0 == 0)
    def _():
        _slab_copy(acc_sh, out0_hbm, sid)

    @pl.when(cid0 == 1)
    def _():
        _slab_copy(acc_sh, out1_hbm, sid)


def _agg(j, i, ex, denp, hflat, zer128, rstride):
    f = pl.kernel(
        functools.partial(_agg_body, rstride),
        out_type=[
            jax.ShapeDtypeStruct((N, 128), jnp.float32),
            jax.ShapeDtypeStruct((N, 128), jnp.float32),
        ],
        mesh=_sc_mesh(),
        compiler_params=pltpu.CompilerParams(needs_layout_passes=False),
        scratch_types=[
            pltpu.VMEM((ACH,), jnp.int32),
            pltpu.VMEM((ACH,), jnp.int32),
            pltpu.VMEM((ACH, 128), jnp.float32),
            pltpu.VMEM((ACH, 128), jnp.float32),
            pltpu.VMEM((GROWS,), jnp.int32),
            pltpu.VMEM((GROWS,), jnp.int32),
            pltpu.VMEM((GROWS, 128), jnp.float32),
            pltpu.VMEM((GROWS, 128), jnp.float32),
            pltpu.VMEM_SHARED((N, 128), jnp.float32),
            pltpu.SemaphoreType.DMA,
            pltpu.SemaphoreType.DMA,
            pltpu.SemaphoreType.DMA,
        ],
    )
    return f(j, i, ex, denp, hflat, zer128)


def _elu(v):
    return jnp.where(v > 0.0, v, jnp.exp(jnp.minimum(v, 0.0)) - 1.0)


def _finish_body(c, o0_ref, o1_ref, b_ref, z_ref):
    v = (o0_ref[:, :c] + o1_ref[:, :c]) * (1.0 / H) + b_ref[0:1, :]
    z_ref[...] = _elu(v)


def _finish(o0, o1, bias, c):
    bn = 1000
    b8 = jnp.broadcast_to(bias[None, :], (8, c))
    return pl.pallas_call(
        functools.partial(_finish_body, c),
        grid=(N // bn,),
        in_specs=[
            pl.BlockSpec((bn, 128), lambda i: (i, 0)),
            pl.BlockSpec((bn, 128), lambda i: (i, 0)),
            pl.BlockSpec((8, c), lambda i: (0, 0)),
        ],
        out_specs=pl.BlockSpec((bn, c), lambda i: (i, 0)),
        out_shape=jax.ShapeDtypeStruct((N, c), jnp.float32),
    )(o0, o1, b8)


def _ae_body(x_ref, w1, b1, g1, s1, w2, b2, g2, s2, dw1, db1, dw2, db2,
             z1_ref, z2_ref, d2_ref):
    inv = 1.0 / math.sqrt(1.0 + 1e-5)
    xv = x_ref[...]
    t1 = _elu(jnp.dot(xv, w1[...], preferred_element_type=jnp.float32) + b1[0:1, :])
    z1 = g1[0:1, :] * (t1 * inv) + s1[0:1, :]
    z1_ref[...] = z1
    t2 = _elu(jnp.dot(z1, w2[...], preferred_element_type=jnp.float32) + b2[0:1, :])
    z2 = g2[0:1, :] * (t2 * inv) + s2[0:1, :]
    z2_ref[...] = z2
    d1 = _elu(jnp.dot(z2, dw1[...], preferred_element_type=jnp.float32) + db1[0:1, :])
    d2 = jnp.dot(d1, dw2[...], preferred_element_type=jnp.float32) + db2[0:1, :]
    d2_ref[...] = jax.nn.sigmoid(d2)


def _ae(x, ae):
    bn = 1000
    c1, c2 = 128, 64

    def row8(v):
        return jnp.broadcast_to(v[None, :], (8, v.shape[0]))

    ws = [
        ae['enc1_w'], row8(ae['enc1_b']), row8(ae['bn1_g']), row8(ae['bn1_b']),
        ae['enc2_w'], row8(ae['enc2_b']), row8(ae['bn2_g']), row8(ae['bn2_b']),
        ae['dec1_w'], row8(ae['dec1_b']), ae['dec2_w'], row8(ae['dec2_b']),
    ]
    specs = [pl.BlockSpec(w.shape, lambda i: (0, 0)) for w in ws]
    return pl.pallas_call(
        _ae_body,
        grid=(N // bn,),
        in_specs=[pl.BlockSpec((bn, c1), lambda i: (i, 0))] + specs,
        out_specs=[
            pl.BlockSpec((bn, c1), lambda i: (i, 0)),
            pl.BlockSpec((bn, c2), lambda i: (i, 0)),
            pl.BlockSpec((bn, c1), lambda i: (i, 0)),
        ],
        out_shape=[
            jax.ShapeDtypeStruct((N, c1), jnp.float32),
            jax.ShapeDtypeStruct((N, c2), jnp.float32),
            jax.ShapeDtypeStruct((N, c1), jnp.float32),
        ],
    )(x, *ws)


def _blockdiag(att):
    h, c = att.shape
    return jnp.reshape(att[:, :, None] * jnp.eye(h, dtype=att.dtype)[:, None, :],
                       (h * c, h))


def _conv(xin, p, j, i, c, zer128):
    w, att_s, att_d = p['W'], p['att_src'], p['att_dst']
    ind = w.shape[0]
    if c == 128:
        w4 = w.reshape(ind, H, 2, CA)
        w = jnp.concatenate(
            [w4[:, :, 0, :].reshape(ind, H * CA), w4[:, :, 1, :].reshape(ind, H * CA)],
            axis=1,
        )
        ss = jnp.concatenate(
            [_blockdiag(att_s[:, :CA]), _blockdiag(att_s[:, CA:])], axis=0
        )
        sd = jnp.concatenate(
            [_blockdiag(att_d[:, :CA]), _blockdiag(att_d[:, CA:])], axis=0
        )
    else:
        ss = _blockdiag(att_s)
        sd = _blockdiag(att_d)
    hfull, attpack, lmax = _prep(xin, w, ss, sd)
    ex, den0, den1 = _att(j, i, attpack, lmax, zer128)
    denp = _denpack(den0, den1)
    rstride = hfull.shape[1] // 128
    hflat = hfull.reshape(N * rstride, 128)
    o0, o1 = _agg(j, i, ex, denp, hflat, zer128, rstride)
    return _finish(o0, o1, p['bias'], c)


def kernel(x, edge_index, params):
    j_in, i_in = edge_index[0], edge_index[1]
    j_out, i_out = edge_index[1], edge_index[0]
    zer128 = jnp.zeros((N, 128), jnp.float32)

    z1i = _conv(x, params['gat_in1'], j_in, i_in, 128, zer128)
    z2i = _conv(z1i, params['gat_in2'], j_in, i_in, 64, zer128)
    z1o = _conv(x, params['gat_out1'], j_out, i_out, 128, zer128)
    z2o = _conv(z1o, params['gat_out2'], j_out, i_out, 64, zer128)
    z1s, z2s, d2 = _ae(x, params['ae'])

    return (
        jnp.concatenate([z1i, z2i], axis=-1),
        jnp.concatenate([z1o, z2o], axis=-1),
        jnp.concatenate([z1s, z2s], axis=-1),
        d2,
    )


# speedup vs baseline: 1.6307x; 1.6307x over previous
import functools
import math

import jax
import jax.numpy as jnp
from jax import lax
from jax.experimental import pallas as pl
from jax.experimental.pallas import tpu as pltpu
from jax.experimental.pallas import tpu_sc as plsc

N = 10000
E = 160000
H = 32
NCORES = 2
NSUB = 16
NW = NCORES * NSUB
CHUNK = 128
NCHUNK = E // CHUNK
RB = 624
TAIL = N - RB * NSUB


@functools.cache
def _sc_mesh():
    return plsc.VectorSubcoreMesh(
        core_axis_name="c", subcore_axis_name="s", num_cores=NCORES, num_subcores=NSUB
    )


_GDN = lax.GatherDimensionNumbers(
    offset_dims=(), collapsed_slice_dims=(0,), start_index_map=(0,)
)


def _splat(vec, idx):
    iv = jnp.full((16, 1), idx, jnp.int32)
    return lax.gather(
        vec, iv, _GDN, (1,), mode=lax.GatherScatterMode.PROMISE_IN_BOUNDS
    )


def _slab_copy(src, dst, sid):
    r0 = sid * RB
    pltpu.sync_copy(src.at[pl.ds(r0, RB)], dst.at[pl.ds(r0, RB)])

    @pl.when(sid == NSUB - 1)
    def _():
        pltpu.sync_copy(src.at[pl.ds(RB * NSUB, TAIL)], dst.at[pl.ds(RB * NSUB, TAIL)])


def _prep_body(x_ref, w_ref, ss_ref, sd_ref, h_ref, ap_ref, l_ref, acc_ref):
    hb = jnp.dot(x_ref[...], w_ref[...], preferred_element_type=jnp.float32)
    h_ref[...] = hb
    asb = jnp.dot(hb, ss_ref[...], preferred_element_type=jnp.float32)
    adb = jnp.dot(hb, sd_ref[...], preferred_element_type=jnp.float32)
    bn = asb.shape[0]
    ap_ref[...] = jnp.concatenate(
        [asb, adb, jnp.zeros((bn, 128 - 2 * H), jnp.float32)], axis=-1
    )
    i = pl.program_id(0)
    nb = pl.num_programs(0)
    ms = jnp.max(asb, axis=0)[None, :]
    md = jnp.max(adb, axis=0)[None, :]
    cur = jnp.broadcast_to(jnp.concatenate([ms, md], axis=-1), (8, 2 * H))

    @pl.when(i == 0)
    def _():
        acc_ref[...] = cur

    @pl.when(i > 0)
    def _():
        acc_ref[...] = jnp.maximum(acc_ref[...], cur)

    @pl.when(i == nb - 1)
    def _():
        s = acc_ref[:, :H] + acc_ref[:, H:]
        l_ref[...] = jnp.where(s >= 0.0, s, 0.2 * s)


def _prep(xin, w, ss, sd):
    n, k = xin.shape
    hc = w.shape[1]
    bn = 1000
    return pl.pallas_call(
        _prep_body,
        grid=(n // bn,),
        in_specs=[
            pl.BlockSpec((bn, k), lambda i: (i, 0)),
            pl.BlockSpec((k, hc), lambda i: (0, 0)),
            pl.BlockSpec((hc, H), lambda i: (0, 0)),
            pl.BlockSpec((hc, H), lambda i: (0, 0)),
        ],
        out_specs=[
            pl.BlockSpec((bn, hc), lambda i: (i, 0)),
            pl.BlockSpec((bn, 128), lambda i: (i, 0)),
            pl.BlockSpec((8, H), lambda i: (0, 0)),
        ],
        out_shape=[
            jax.ShapeDtypeStruct((n, hc), jnp.float32),
            jax.ShapeDtypeStruct((n, 128), jnp.float32),
            jax.ShapeDtypeStruct((8, H), jnp.float32),
        ],
        scratch_shapes=[pltpu.VMEM((8, 2 * H), jnp.float32)],
    )(xin, w, ss, sd)


def _att_body(
    j_hbm, i_hbm, ap_hbm, l_hbm, z_hbm,
    ex_hbm, den0_hbm, den1_hbm,
    jbuf, ibuf, apj, api, exb, lb, den_sh, sem,
):
    cid0 = lax.axis_index("c")
    sid = lax.axis_index("s")
    wid = cid0 * NSUB + sid
    _slab_copy(z_hbm, den_sh, sid)
    pltpu.sync_copy(l_hbm, lb)
    plsc.subcore_barrier()
    l0 = lb[0, 0:16]
    l1 = lb[0, 16:32]
    zero = jnp.zeros((16,), jnp.float32)

    def zrow(k, c2):
        for s in range(2, 8):
            exb[k, s * 16:(s + 1) * 16] = zero
        return c2

    lax.fori_loop(0, CHUNK, zrow, 0)

    def chunk_body(t, carry):
        base = (wid + t * NW) * CHUNK
        pltpu.sync_copy(j_hbm.at[pl.ds(base, CHUNK)], jbuf)
        pltpu.sync_copy(i_hbm.at[pl.ds(base, CHUNK)], ibuf)
        pltpu.async_copy(ap_hbm.at[jbuf], apj, sem).wait()
        pltpu.async_copy(ap_hbm.at[ibuf], api, sem).wait()

        def row(k, c2):
            s0 = apj[k, 0:16] + api[k, 32:48]
            s1 = apj[k, 16:32] + api[k, 48:64]
            exb[k, 0:16] = jnp.exp(jnp.where(s0 >= 0.0, s0, 0.2 * s0) - l0)
            exb[k, 16:32] = jnp.exp(jnp.where(s1 >= 0.0, s1, 0.2 * s1) - l1)
            return c2

        lax.fori_loop(0, CHUNK, row, 0)
        pltpu.sync_copy(exb, ex_hbm.at[pl.ds(base, CHUNK)])
        pltpu.sync_copy(exb, den_sh.at[ibuf], add=True)
        return carry

    nt = (NCHUNK - wid + NW - 1) // NW
    lax.fori_loop(0, nt, chunk_body, 0)
    plsc.subcore_barrier()

    @pl.when(cid0 == 0)
    def _():
        _slab_copy(den_sh, den0_hbm, sid)

    @pl.when(cid0 == 1)
    def _():
        _slab_copy(den_sh, den1_hbm, sid)


def _att(j, i, attpack, lmax, zer128):
    f = pl.kernel(
        _att_body,
        out_type=[
            jax.ShapeDtypeStruct((E, 128), jnp.float32),
            jax.ShapeDtypeStruct((N, 128), jnp.float32),
            jax.ShapeDtypeStruct((N, 128), jnp.float32),
        ],
        mesh=_sc_mesh(),
        compiler_params=pltpu.CompilerParams(needs_layout_passes=False),
        scratch_types=[
            pltpu.VMEM((CHUNK,), jnp.int32),
            pltpu.VMEM((CHUNK,), jnp.int32),
            pltpu.VMEM((CHUNK, 128), jnp.float32),
            pltpu.VMEM((CHUNK, 128), jnp.float32),
            pltpu.VMEM((CHUNK, 128), jnp.float32),
            pltpu.VMEM((8, H), jnp.float32),
            pltpu.VMEM_SHARED((N, 128), jnp.float32),
            pltpu.SemaphoreType.DMA,
        ],
    )
    return f(j, i, attpack, lmax, zer128)


def _denpack_body(d0_ref, d1_ref, dp_ref):
    dp_ref[...] = d0_ref[...] + d1_ref[...]


def _denpack(d0, d1):
    bn = 1000
    return pl.pallas_call(
        _denpack_body,
        grid=(N // bn,),
        in_specs=[
            pl.BlockSpec((bn, 128), lambda i: (i, 0)),
            pl.BlockSpec((bn, 128), lambda i: (i, 0)),
        ],
        out_specs=pl.BlockSpec((bn, 128), lambda i: (i, 0)),
        out_shape=jax.ShapeDtypeStruct((N, 128), jnp.float32),
    )(d0, d1)


GROWS = 64
ACH = 64
NACH = E // ACH
CA = 64


def _ig16(val):
    return jnp.zeros((16,), jnp.int32) + val


def _agg_body(
    rstride,
    j_hbm, i_hbm, ex_hbm, denp_hbm, h_hbm, z_hbm,
    out0_hbm, out1_hbm,
    jbuf, ibuf, ewb, dmb, idx0, idx1, gb0, gb1, acc_sh, semd, sem0, sem1,
):
    cid0 = lax.axis_index("c")
    sid = lax.axis_index("s")
    wid = cid0 * NSUB + sid
    _slab_copy(z_hbm, acc_sh, sid)
    plsc.subcore_barrier()
    iota = lax.iota(jnp.int32, 16)
    halves = rstride // 16
    rowspe = 16 * halves
    eps = GROWS // rowspe
    nsubs = ACH // eps
    zero = jnp.zeros((16,), jnp.float32)
    idxs = [idx0, idx1]
    gbs = [gb0, gb1]
    sems = [sem0, sem1]

    def build(q, ib):
        for e in range(eps):
            js = plsc.load_gather(jbuf, [_ig16(q * eps + e)])
            b0 = js * rstride + iota
            ib[pl.ds(e * rowspe, 16)] = b0
            if halves == 2:
                ib[pl.ds(e * rowspe + 16, 16)] = b0 + 16

    def fma(q, gb):
        for e in range(eps):
            k = q * eps + e
            w0 = ewb[k, 0:16]
            w1 = ewb[k, 16:32]
            acc = [zero] * 8
            for p in range(16):
                wsrc0 = w0 if 2 * p < 16 else w1
                wsrc1 = w0 if 2 * p + 1 < 16 else w1
                ws0 = _splat(wsrc0, (2 * p) % 16)
                ws1 = _splat(wsrc1, (2 * p + 1) % 16)
                ra = e * rowspe + p
                for s in range(4):
                    acc[s] = acc[s] + ws0 * gb[ra, s * 16:(s + 1) * 16]
                    acc[s] = acc[s] + ws1 * gb[ra, CA + s * 16:CA + (s + 1) * 16]
                if halves == 2:
                    rb = ra + 16
                    for s in range(4):
                        acc[4 + s] = acc[4 + s] + ws0 * gb[rb, s * 16:(s + 1) * 16]
                        acc[4 + s] = acc[4 + s] + ws1 * gb[rb, CA + s * 16:CA + (s + 1) * 16]
            for s in range(8):
                dmb[k, s * 16:(s + 1) * 16] = acc[s]

    def chunk_body(t, carry):
        base = (wid + t * NW) * ACH
        pltpu.sync_copy(j_hbm.at[pl.ds(base, ACH)], jbuf)
        pltpu.sync_copy(i_hbm.at[pl.ds(base, ACH)], ibuf)
        den_cp = pltpu.async_copy(denp_hbm.at[ibuf], dmb, semd)
        pltpu.sync_copy(ex_hbm.at[pl.ds(base, ACH)], ewb)
        den_cp.wait()

        def wrow(k, c2):
            ewb[k, 0:16] = ewb[k, 0:16] / (dmb[k, 0:16] + 1e-16)
            ewb[k, 16:32] = ewb[k, 16:32] / (dmb[k, 16:32] + 1e-16)
            return c2

        lax.fori_loop(0, ACH, wrow, 0)

        def pair(q2, c3):
            q = q2 * 2
            build(q, idx0)
            cp0 = pltpu.async_copy(h_hbm.at[idx0], gb0, sem0)
            build(q + 1, idx1)
            cp1 = pltpu.async_copy(h_hbm.at[idx1], gb1, sem1)
            cp0.wait()
            fma(q, gb0)
            cp1.wait()
            fma(q + 1, gb1)
            return c3

        lax.fori_loop(0, nsubs // 2, pair, 0)
        pltpu.sync_copy(dmb, acc_sh.at[ibuf], add=True)
        return carry

    nt = (NACH - wid + NW - 1) // NW
    lax.fori_loop(0, nt, chunk_body, 0)
    plsc.subcore_barrier()

    @pl.when(cid0 == 0)
    def _():
        _slab_copy(acc_sh, out0_hbm, sid)

    @pl.when(cid0 == 1)
    def _():
        _slab_copy(acc_sh, out1_hbm, sid)


def _agg(j, i, ex, denp, hflat, zer128, rstride):
    f = pl.kernel(
        functools.partial(_agg_body, rstride),
        out_type=[
            jax.ShapeDtypeStruct((N, 128), jnp.float32),
            jax.ShapeDtypeStruct((N, 128), jnp.float32),
        ],
        mesh=_sc_mesh(),
        compiler_params=pltpu.CompilerParams(needs_layout_passes=False),
        scratch_types=[
            pltpu.VMEM((ACH,), jnp.int32),
            pltpu.VMEM((ACH,), jnp.int32),
            pltpu.VMEM((ACH, 128), jnp.float32),
            pltpu.VMEM((ACH, 128), jnp.float32),
            pltpu.VMEM((GROWS,), jnp.int32),
            pltpu.VMEM((GROWS,), jnp.int32),
            pltpu.VMEM((GROWS, 128), jnp.float32),
            pltpu.VMEM((GROWS, 128), jnp.float32),
            pltpu.VMEM_SHARED((N, 128), jnp.float32),
            pltpu.SemaphoreType.DMA,
            pltpu.SemaphoreType.DMA,
            pltpu.SemaphoreType.DMA,
        ],
    )
    return f(j, i, ex, denp, hflat, zer128)


def _elu(v):
    return jnp.where(v > 0.0, v, jnp.exp(jnp.minimum(v, 0.0)) - 1.0)


def _finish_body(c, o0_ref, o1_ref, b_ref, z_ref):
    v = (o0_ref[:, :c] + o1_ref[:, :c]) * (1.0 / H) + b_ref[0:1, :]
    z_ref[...] = _elu(v)


def _finish(o0, o1, bias, c):
    bn = 1000
    b8 = jnp.broadcast_to(bias[None, :], (8, c))
    return pl.pallas_call(
        functools.partial(_finish_body, c),
        grid=(N // bn,),
        in_specs=[
            pl.BlockSpec((bn, 128), lambda i: (i, 0)),
            pl.BlockSpec((bn, 128), lambda i: (i, 0)),
            pl.BlockSpec((8, c), lambda i: (0, 0)),
        ],
        out_specs=pl.BlockSpec((bn, c), lambda i: (i, 0)),
        out_shape=jax.ShapeDtypeStruct((N, c), jnp.float32),
    )(o0, o1, b8)


def _ae_body(x_ref, w1, b1, g1, s1, w2, b2, g2, s2, dw1, db1, dw2, db2,
             z1_ref, z2_ref, d2_ref):
    inv = 1.0 / math.sqrt(1.0 + 1e-5)
    xv = x_ref[...]
    t1 = _elu(jnp.dot(xv, w1[...], preferred_element_type=jnp.float32) + b1[0:1, :])
    z1 = g1[0:1, :] * (t1 * inv) + s1[0:1, :]
    z1_ref[...] = z1
    t2 = _elu(jnp.dot(z1, w2[...], preferred_element_type=jnp.float32) + b2[0:1, :])
    z2 = g2[0:1, :] * (t2 * inv) + s2[0:1, :]
    z2_ref[...] = z2
    d1 = _elu(jnp.dot(z2, dw1[...], preferred_element_type=jnp.float32) + db1[0:1, :])
    d2 = jnp.dot(d1, dw2[...], preferred_element_type=jnp.float32) + db2[0:1, :]
    d2_ref[...] = jax.nn.sigmoid(d2)


def _ae(x, ae):
    bn = 1000
    c1, c2 = 128, 64

    def row8(v):
        return jnp.broadcast_to(v[None, :], (8, v.shape[0]))

    ws = [
        ae['enc1_w'], row8(ae['enc1_b']), row8(ae['bn1_g']), row8(ae['bn1_b']),
        ae['enc2_w'], row8(ae['enc2_b']), row8(ae['bn2_g']), row8(ae['bn2_b']),
        ae['dec1_w'], row8(ae['dec1_b']), ae['dec2_w'], row8(ae['dec2_b']),
    ]
    specs = [pl.BlockSpec(w.shape, lambda i: (0, 0)) for w in ws]
    return pl.pallas_call(
        _ae_body,
        grid=(N // bn,),
        in_specs=[pl.BlockSpec((bn, c1), lambda i: (i, 0))] + specs,
        out_specs=[
            pl.BlockSpec((bn, c1), lambda i: (i, 0)),
            pl.BlockSpec((bn, c2), lambda i: (i, 0)),
            pl.BlockSpec((bn, c1), lambda i: (i, 0)),
        ],
        out_shape=[
            jax.ShapeDtypeStruct((N, c1), jnp.float32),
            jax.ShapeDtypeStruct((N, c2), jnp.float32),
            jax.ShapeDtypeStruct((N, c1), jnp.float32),
        ],
    )(x, *ws)


def _blockdiag(att):
    h, c = att.shape
    return jnp.reshape(att[:, :, None] * jnp.eye(h, dtype=att.dtype)[:, None, :],
                       (h * c, h))


def _conv(xin, p, j, i, c, zer128):
    w, att_s, att_d = p['W'], p['att_src'], p['att_dst']
    ind = w.shape[0]
    if c == 128:
        w4 = w.reshape(ind, H, 2, CA)
        w = jnp.concatenate(
            [w4[:, :, 0, :].reshape(ind, H * CA), w4[:, :, 1, :].reshape(ind, H * CA)],
            axis=1,
        )
        ss = jnp.concatenate(
            [_blockdiag(att_s[:, :CA]), _blockdiag(att_s[:, CA:])], axis=0
        )
        sd = jnp.concatenate(
            [_blockdiag(att_d[:, :CA]), _blockdiag(att_d[:, CA:])], axis=0
        )
    else:
        ss = _blockdiag(att_s)
        sd = _blockdiag(att_d)
    hfull, attpack, lmax = _prep(xin, w, ss, sd)
    ex, den0, den1 = _att(j, i, attpack, lmax, zer128)
    denp = _denpack(den0, den1)
    rstride = hfull.shape[1] // 128
    hflat = hfull.reshape(N * rstride, 128)
    o0, o1 = _agg(j, i, ex, denp, hflat, zer128, rstride)
    return _finish(o0, o1, p['bias'], c)


def kernel(x, edge_index, params):
    j_in, i_in = edge_index[0], edge_index[1]
    j_out, i_out = edge_index[1], edge_index[0]
    zer128 = jnp.zeros((N, 128), jnp.float32)

    z1i = _conv(x, params['gat_in1'], j_in, i_in, 128, zer128)
    z2i = _conv(z1i, params['gat_in2'], j_in, i_in, 64, zer128)
    z1o = _conv(x, params['gat_out1'], j_out, i_out, 128, zer128)
    z2o = _conv(z1o, params['gat_out2'], j_out, i_out, 64, zer128)
    z1s, z2s, d2 = _ae(x, params['ae'])

    return (
        jnp.concatenate([z1i, z2i], axis=-1),
        jnp.concatenate([z1o, z2o], axis=-1),
        jnp.concatenate([z1s, z2s], axis=-1),
        d2,
    )
